# baseline (speedup 1.0000x reference)
import functools

import jax
import jax.numpy as jnp
from jax import lax
from jax.experimental import pallas as pl
from jax.experimental.pallas import tpu as pltpu
from jax.experimental.pallas import tpu_sc as plsc

_NC, _NS = 2, 16
_CH = 32
_LANES = 16


def _build_sc_kernel(B, LV, C, L, Lp):
    TROWS = LV + 1
    n_chunks = L // _CH
    per_half = n_chunks // 2
    tail = n_chunks * _CH
    mesh = plsc.VectorSubcoreMesh(core_axis_name="c", subcore_axis_name="s")

    @functools.partial(
        pl.kernel,
        mesh=mesh,
        out_type=jax.ShapeDtypeStruct((B, L * C), jnp.float32),
        scratch_types=[
            pltpu.VMEM((Lp,), jnp.int32),
            pltpu.VMEM((Lp,), jnp.int32),
            pltpu.VMEM((_CH, C // 2), jnp.int32),
            pltpu.VMEM((_CH, C // 2), jnp.int32),
            pltpu.VMEM((_CH * C,), jnp.float32),
            pltpu.VMEM((_CH, C // 2), jnp.int32),
            pltpu.VMEM((_CH, C // 2), jnp.int32),
            pltpu.VMEM((_CH * C,), jnp.float32),
            pltpu.VMEM((8, C // 2), jnp.int32),
            pltpu.VMEM((8, C // 2), jnp.int32),
            pltpu.VMEM((8 * C,), jnp.float32),
            pltpu.SemaphoreType.DMA,
            pltpu.SemaphoreType.DMA,
            pltpu.SemaphoreType.DMA,
            pltpu.SemaphoreType.DMA,
            pltpu.SemaphoreType.DMA,
            pltpu.SemaphoreType.DMA,
            pltpu.SemaphoreType.DMA,
            pltpu.SemaphoreType.DMA,
        ],
    )
    def sc_kernel(x_hbm, pol_hbm, pe_hbm, out_hbm, pol_v, idx_v,
                  pe0, g0, r0, pe1, g1, r1, pe_t, g_t, r_t,
                  spe0, sg0, so0, spe1, sg1, so1, spt, sgt):
        wid = lax.axis_index("s") * _NC + lax.axis_index("c")
        b = wid // 2
        h = wid % 2

        pltpu.sync_copy(pol_hbm.at[b], pol_v)

        sets = ((pe0, g0, r0, spe0, sg0, so0), (pe1, g1, r1, spe1, sg1, so1))
        starts = [(h * per_half + k) * _CH for k in range(per_half)]

        def issue_pe(k):
            pe_b, _, _, spe, _, _ = sets[k % 2]
            return pltpu.async_copy(pe_hbm.at[pl.ds(starts[k], _CH)], pe_b,
                                    spe)

        pe_pre = issue_pe(0)

        bbase = b * TROWS
        lanes = lax.iota(jnp.int32, _LANES)
        lanes2 = lanes * 2

        def _take(v, i):
            return v.at[i].get(mode="promise_in_bounds")

        def scan_body(i, carry):
            ch = pol_v[pl.ds(i * _LANES, _LANES)]
            cs = ch
            for d in (1, 2, 4, 8):
                shifted = _take(cs, jnp.maximum(lanes - d, 0))
                cs = cs + jnp.where(lanes >= d, shifted, 0)
            excl = cs - ch + carry
            idx_v[pl.ds(i * _LANES, _LANES)] = bbase + LV + ch * (excl - LV)
            return carry + _take(cs, jnp.full((_LANES,), _LANES - 1, jnp.int32))

        lax.fori_loop(0, Lp // _LANES, scan_body,
                      jnp.zeros((_LANES,), jnp.int32))

        def issue_g(k):
            _, g_b, _, _, sg, _ = sets[k % 2]
            return pltpu.async_copy(x_hbm.at[idx_v.at[pl.ds(starts[k], _CH)]],
                                    g_b, sg)

        HI = jnp.int32(-65536)
        lanes_half = lanes >> 1
        lanes_half8 = lanes_half + 8
        even_lane = (lanes & 1) == 0

        def _f32(w):
            return lax.bitcast_convert_type(w, jnp.float32)

        def sum_rows(pe_b, g_b, res, nrows):
            def row_body(r, _):
                rowbase = r * C

                def q_body(q, _2):
                    base = rowbase + q * (8 * 2 * _LANES)
                    for t in range(8):
                        sl = pl.ds(q * (8 * _LANES) + t * _LANES, _LANES)
                        pw = pe_b[r, sl]
                        gw = g_b[r, sl]
                        se = _f32(pw << 16) + _f32(gw << 16)
                        so_ = _f32(pw & HI) + _f32(gw & HI)
                        o = base + t * 2 * _LANES
                        res[pl.ds(o, _LANES)] = jnp.where(
                            even_lane, _take(se, lanes_half),
                            _take(so_, lanes_half))
                        res[pl.ds(o + _LANES, _LANES)] = jnp.where(
                            even_lane, _take(se, lanes_half8),
                            _take(so_, lanes_half8))
                    return 0

                lax.fori_loop(0, C // (8 * 2 * _LANES), q_body, 0)
                return 0

            lax.fori_loop(0, nrows, row_body, 0)

        out_inflight = [None, None]
        in_h = (pe_pre, issue_g(0))
        for k in range(per_half):
            if k + 1 < per_half:
                nxt = (k + 1) % 2
                if out_inflight[nxt] is not None:
                    out_inflight[nxt].wait()
                    out_inflight[nxt] = None
                nxt_h = (issue_pe(k + 1), issue_g(k + 1))
            in_h[0].wait()
            in_h[1].wait()
            pe_b, g_b, res, _, _, so = sets[k % 2]
            sum_rows(pe_b, g_b, res, _CH)
            out_inflight[k % 2] = pltpu.async_copy(
                res, out_hbm.at[b, pl.ds(starts[k] * C, _CH * C)], so)
            if k + 1 < per_half:
                in_h = nxt_h
        for o in out_inflight:
            if o is not None:
                o.wait()

        @pl.when(h == 1)
        def _():
            c1 = pltpu.async_copy(pe_hbm.at[pl.ds(tail, 8)], pe_t, spt)
            c2 = pltpu.async_copy(x_hbm.at[idx_v.at[pl.ds(tail, 8)]], g_t, sgt)
            c1.wait()
            c2.wait()
            sum_rows(pe_t, g_t, r_t, 1)
            pltpu.sync_copy(r_t.at[pl.ds(0, C)],
                            out_hbm.at[b, pl.ds(tail * C, C)])

    return sc_kernel


def kernel(x, policy, mask_token, pos_embed):
    B, LV, C = x.shape
    L = policy.shape[1]
    Lp = ((L + _LANES - 1) // _LANES) * _LANES

    polp = jnp.pad(policy.astype(jnp.int32), ((0, 0), (0, Lp - L)))
    x_ext = jnp.concatenate(
        [x, jnp.broadcast_to(mask_token, (B, 1, C))], axis=1
    ).reshape(B * (LV + 1), C).astype(jnp.bfloat16)
    x_ext = lax.bitcast_convert_type(
        x_ext.reshape(B * (LV + 1), C // 2, 2), jnp.int32)
    pe = jnp.pad(pos_embed.reshape(L, C), ((0, 7), (0, 0))).astype(
        jnp.bfloat16)
    pe = lax.bitcast_convert_type(pe.reshape(L + 7, C // 2, 2), jnp.int32)

    sc = _build_sc_kernel(B, LV, C, L, Lp)
    return sc(x_ext, polp, pe).reshape(B, L, C)


# speedup vs baseline: 1.5237x; 1.5237x over previous
import functools

import jax
import jax.numpy as jnp
from jax import lax
from jax.experimental import pallas as pl
from jax.experimental.pallas import tpu as pltpu
from jax.experimental.pallas import tpu_sc as plsc

_NC, _NS = 2, 16
_CH = 32
_LANES = 16


def _build_sc_kernel(B, LV, C, L, Lp):
    TROWS = LV + 1
    n_chunks = L // _CH
    per_half = n_chunks // 2
    tail = n_chunks * _CH
    mesh = plsc.VectorSubcoreMesh(core_axis_name="c", subcore_axis_name="s")

    @functools.partial(
        pl.kernel,
        mesh=mesh,
        out_type=jax.ShapeDtypeStruct((B, L * C), jnp.float32),
        scratch_types=[
            pltpu.VMEM((Lp,), jnp.int32),
            pltpu.VMEM((Lp,), jnp.int32),
            pltpu.VMEM((_CH, C // 2), jnp.int32),
            pltpu.VMEM((_CH, C // 2), jnp.int32),
            pltpu.VMEM((_CH * C,), jnp.float32),
            pltpu.VMEM((_CH, C // 2), jnp.int32),
            pltpu.VMEM((_CH, C // 2), jnp.int32),
            pltpu.VMEM((_CH * C,), jnp.float32),
            pltpu.VMEM((8, C // 2), jnp.int32),
            pltpu.VMEM((8, C // 2), jnp.int32),
            pltpu.VMEM((8 * C,), jnp.float32),
            pltpu.SemaphoreType.DMA,
            pltpu.SemaphoreType.DMA,
            pltpu.SemaphoreType.DMA,
            pltpu.SemaphoreType.DMA,
            pltpu.SemaphoreType.DMA,
            pltpu.SemaphoreType.DMA,
            pltpu.SemaphoreType.DMA,
            pltpu.SemaphoreType.DMA,
        ],
    )
    def sc_kernel(x_hbm, pol_hbm, pe_hbm, out_hbm, pol_v, idx_v,
                  pe0, g0, r0, pe1, g1, r1, pe_t, g_t, r_t,
                  spe0, sg0, so0, spe1, sg1, so1, spt, sgt):
        wid = lax.axis_index("s") * _NC + lax.axis_index("c")
        b = wid // 2
        h = wid % 2

        pltpu.sync_copy(pol_hbm.at[b], pol_v)

        sets = ((pe0, g0, r0, spe0, sg0, so0), (pe1, g1, r1, spe1, sg1, so1))
        starts = [(h * per_half + k) * _CH for k in range(per_half)]

        def issue_pe(k):
            pe_b, _, _, spe, _, _ = sets[k % 2]
            return pltpu.async_copy(pe_hbm.at[pl.ds(starts[k], _CH)], pe_b,
                                    spe)

        pe_pre = issue_pe(0)

        bbase = b * TROWS
        lanes = lax.iota(jnp.int32, _LANES)
        lanes2 = lanes * 2

        def _take(v, i):
            return v.at[i].get(mode="promise_in_bounds")

        def scan_body(i, carry):
            ch = pol_v[pl.ds(i * _LANES, _LANES)]
            cs = ch
            for d in (1, 2, 4, 8):
                shifted = _take(cs, jnp.maximum(lanes - d, 0))
                cs = cs + jnp.where(lanes >= d, shifted, 0)
            excl = cs - ch + carry
            idx_v[pl.ds(i * _LANES, _LANES)] = bbase + LV + ch * (excl - LV)
            return carry + _take(cs, jnp.full((_LANES,), _LANES - 1, jnp.int32))

        lax.fori_loop(0, Lp // _LANES, scan_body,
                      jnp.zeros((_LANES,), jnp.int32))

        def issue_g(k):
            _, g_b, _, _, sg, _ = sets[k % 2]
            return pltpu.async_copy(x_hbm.at[idx_v.at[pl.ds(starts[k], _CH)]],
                                    g_b, sg)

        HI = jnp.int32(-65536)

        def _f32(w):
            return lax.bitcast_convert_type(w, jnp.float32)

        def sum_rows(pe_b, g_b, res, nrows):
            def row_body(r, _):
                rowbase = r * C

                def q_body(q, _2):
                    base = rowbase + q * (8 * 2 * _LANES)
                    for t in range(8):
                        sl = pl.ds(q * (8 * _LANES) + t * _LANES, _LANES)
                        pw = pe_b[r, sl]
                        gw = g_b[r, sl]
                        o = base + t * 2 * _LANES
                        res[pl.ds(o, _LANES)] = (
                            _f32(pw << 16) + _f32(gw << 16))
                        res[pl.ds(o + _LANES, _LANES)] = (
                            _f32(pw & HI) + _f32(gw & HI))
                    return 0

                lax.fori_loop(0, C // (8 * 2 * _LANES), q_body, 0)
                return 0

            lax.fori_loop(0, nrows, row_body, 0)

        out_inflight = [None, None]
        in_h = (pe_pre, issue_g(0))
        for k in range(per_half):
            if k + 1 < per_half:
                nxt = (k + 1) % 2
                if out_inflight[nxt] is not None:
                    out_inflight[nxt].wait()
                    out_inflight[nxt] = None
                nxt_h = (issue_pe(k + 1), issue_g(k + 1))
            in_h[0].wait()
            in_h[1].wait()
            pe_b, g_b, res, _, _, so = sets[k % 2]
            sum_rows(pe_b, g_b, res, _CH)
            out_inflight[k % 2] = pltpu.async_copy(
                res, out_hbm.at[b, pl.ds(starts[k] * C, _CH * C)], so)
            if k + 1 < per_half:
                in_h = nxt_h
        for o in out_inflight:
            if o is not None:
                o.wait()

        @pl.when(h == 1)
        def _():
            c1 = pltpu.async_copy(pe_hbm.at[pl.ds(tail, 8)], pe_t, spt)
            c2 = pltpu.async_copy(x_hbm.at[idx_v.at[pl.ds(tail, 8)]], g_t, sgt)
            c1.wait()
            c2.wait()
            sum_rows(pe_t, g_t, r_t, 1)
            pltpu.sync_copy(r_t.at[pl.ds(0, C)],
                            out_hbm.at[b, pl.ds(tail * C, C)])

    return sc_kernel


def kernel(x, policy, mask_token, pos_embed):
    B, LV, C = x.shape
    L = policy.shape[1]
    Lp = ((L + _LANES - 1) // _LANES) * _LANES

    polp = jnp.pad(policy.astype(jnp.int32), ((0, 0), (0, Lp - L)))
    def _pack(a):
        r_, c_ = a.shape
        ab = a.astype(jnp.bfloat16).reshape(r_, c_ // 32, 2, _LANES)
        ab = ab.transpose(0, 1, 3, 2)
        return lax.bitcast_convert_type(ab, jnp.int32).reshape(r_, c_ // 2)

    x_ext = _pack(jnp.concatenate(
        [x, jnp.broadcast_to(mask_token, (B, 1, C))], axis=1
    ).reshape(B * (LV + 1), C))
    pe = _pack(jnp.pad(pos_embed.reshape(L, C), ((0, 7), (0, 0))))

    sc = _build_sc_kernel(B, LV, C, L, Lp)
    return sc(x_ext, polp, pe).reshape(B, L, C)


# speedup vs baseline: 1.5778x; 1.0355x over previous
import functools

import jax
import jax.numpy as jnp
from jax import lax
from jax.experimental import pallas as pl
from jax.experimental.pallas import tpu as pltpu
from jax.experimental.pallas import tpu_sc as plsc

_NC, _NS = 2, 16
_CH = 32
_LANES = 16


def _build_sc_kernel(B, LV, C, L, Lp):
    TROWS = LV + 1
    TROWS_P = ((TROWS + 7) // 8) * 8
    Cw = C // 2
    n_chunks = L // _CH
    per_half = n_chunks // 2
    tail = n_chunks * _CH
    PH = 14
    mesh = plsc.VectorSubcoreMesh(core_axis_name="c", subcore_axis_name="s")

    @functools.partial(
        pl.kernel,
        mesh=mesh,
        out_type=(jax.ShapeDtypeStruct((B, L * C), jnp.float32),
                  jax.ShapeDtypeStruct((B * TROWS_P, Cw), jnp.int32)),
        scratch_types=[
            pltpu.VMEM((Lp,), jnp.int32),
            pltpu.VMEM((Lp,), jnp.int32),
            pltpu.VMEM((_CH, Cw), jnp.int32),
            pltpu.VMEM((_CH, Cw), jnp.int32),
            pltpu.VMEM((_CH * C,), jnp.float32),
            pltpu.VMEM((_CH, Cw), jnp.int32),
            pltpu.VMEM((_CH, Cw), jnp.int32),
            pltpu.VMEM((_CH * C,), jnp.float32),
            pltpu.VMEM((8, Cw), jnp.int32),
            pltpu.VMEM((8, Cw), jnp.int32),
            pltpu.VMEM((8 * C,), jnp.float32),
            pltpu.SemaphoreType.DMA,
            pltpu.SemaphoreType.DMA,
            pltpu.SemaphoreType.DMA,
            pltpu.SemaphoreType.DMA,
            pltpu.SemaphoreType.DMA,
            pltpu.SemaphoreType.DMA,
            pltpu.SemaphoreType.DMA,
            pltpu.SemaphoreType.DMA,
        ],
    )
    def sc_kernel(x_hbm, mt_hbm, pol_hbm, pe_hbm, out_hbm, xp_hbm,
                  pol_v, idx_v,
                  pe0, g0, r0, pe1, g1, r1, pe_t, g_t, r_t,
                  spe0, sg0, so0, spe1, sg1, so1, spt, sgt):
        c = lax.axis_index("c")
        s = lax.axis_index("s")
        b = c * (_NS // 2) + s // 2
        h = s % 2
        bbase = b * TROWS_P

        HI = jnp.int32(-65536)
        RND = jnp.int32(32768)
        lanes = lax.iota(jnp.int32, _LANES)

        def _i32(v):
            return lax.bitcast_convert_type(v, jnp.int32)

        def _f32(w):
            return lax.bitcast_convert_type(w, jnp.float32)

        xin, xpk = r0, pe0

        def pack_to(r, src_base):
            def pq(q, _2):
                for t in range(4):
                    g2 = q * 4 + t
                    lo = _i32(xin[pl.ds(src_base + g2 * 32, _LANES)])
                    hi = _i32(xin[pl.ds(src_base + g2 * 32 + _LANES,
                                        _LANES)])
                    xpk[r, pl.ds(g2 * _LANES, _LANES)] = (
                        lax.shift_right_logical(lo + RND, 16)
                        | ((hi + RND) & HI))
                return 0

            lax.fori_loop(0, C // 128, pq, 0)

        def pack_rows(src_off, dst_off, nrows):
            pltpu.async_copy(x_hbm.at[pl.ds(src_off * C, nrows * C)],
                             xin.at[pl.ds(0, nrows * C)], spt).wait()

            def prow(r, _):
                pack_to(r, r * C)
                return 0

            lax.fori_loop(0, nrows, prow, 0)
            pltpu.sync_copy(xpk.at[pl.ds(0, nrows)],
                            xp_hbm.at[pl.ds(dst_off, nrows)])

        pstart = h * (8 * ((LV - PH * _CH) // 8))
        for k in range(PH):
            off = pstart + k * _CH
            pack_rows(b * LV + off, bbase + off, _CH)

        @pl.when(h == 1)
        def _():
            pltpu.async_copy(x_hbm.at[pl.ds((b * LV + LV - 1) * C, C)],
                             xin.at[pl.ds(0, C)], spt).wait()
            pltpu.async_copy(mt_hbm, xin.at[pl.ds(C, C)], sgt).wait()
            pack_to(0, 0)
            pack_to(1, C)
            pltpu.sync_copy(xpk.at[pl.ds(0, 8)],
                            xp_hbm.at[pl.ds(bbase + LV - 1, 8)])

        pltpu.sync_copy(pol_hbm.at[b], pol_v)
        plsc.subcore_barrier()

        sets = ((pe0, g0, r0, spe0, sg0, so0), (pe1, g1, r1, spe1, sg1, so1))
        starts = [(h * per_half + k) * _CH for k in range(per_half)]

        def issue_pe(k):
            pe_b, _, _, spe, _, _ = sets[k % 2]
            return pltpu.async_copy(pe_hbm.at[pl.ds(starts[k], _CH)], pe_b,
                                    spe)

        pe_pre = issue_pe(0)

        def _take(v, i):
            return v.at[i].get(mode="promise_in_bounds")

        def scan_body(i, carry):
            ch = pol_v[pl.ds(i * _LANES, _LANES)]
            cs = ch
            for d in (1, 2, 4, 8):
                shifted = _take(cs, jnp.maximum(lanes - d, 0))
                cs = cs + jnp.where(lanes >= d, shifted, 0)
            excl = cs - ch + carry
            idx_v[pl.ds(i * _LANES, _LANES)] = bbase + LV + ch * (excl - LV)
            return carry + _take(cs, jnp.full((_LANES,), _LANES - 1, jnp.int32))

        lax.fori_loop(0, Lp // _LANES, scan_body,
                      jnp.zeros((_LANES,), jnp.int32))

        def issue_g(k):
            _, g_b, _, _, sg, _ = sets[k % 2]
            return pltpu.async_copy(xp_hbm.at[idx_v.at[pl.ds(starts[k], _CH)]],
                                    g_b, sg)

        def sum_rows(pe_b, g_b, res, nrows):
            def row_body(r, _):
                rowbase = r * C

                def q_body(q, _2):
                    base = rowbase + q * (8 * 2 * _LANES)
                    for t in range(8):
                        sl = pl.ds(q * (8 * _LANES) + t * _LANES, _LANES)
                        pw = pe_b[r, sl]
                        gw = g_b[r, sl]
                        o = base + t * 2 * _LANES
                        res[pl.ds(o, _LANES)] = (
                            _f32(pw << 16) + _f32(gw << 16))
                        res[pl.ds(o + _LANES, _LANES)] = (
                            _f32(pw & HI) + _f32(gw & HI))
                    return 0

                lax.fori_loop(0, C // (8 * 2 * _LANES), q_body, 0)
                return 0

            lax.fori_loop(0, nrows, row_body, 0)

        out_inflight = [None, None]
        in_h = (pe_pre, issue_g(0))
        for k in range(per_half):
            if k + 1 < per_half:
                nxt = (k + 1) % 2
                if out_inflight[nxt] is not None:
                    out_inflight[nxt].wait()
                    out_inflight[nxt] = None
                nxt_h = (issue_pe(k + 1), issue_g(k + 1))
            in_h[0].wait()
            in_h[1].wait()
            pe_b, g_b, res, _, _, so = sets[k % 2]
            sum_rows(pe_b, g_b, res, _CH)
            out_inflight[k % 2] = pltpu.async_copy(
                res, out_hbm.at[b, pl.ds(starts[k] * C, _CH * C)], so)
            if k + 1 < per_half:
                in_h = nxt_h
        for o in out_inflight:
            if o is not None:
                o.wait()

        @pl.when(h == 1)
        def _():
            c1 = pltpu.async_copy(pe_hbm.at[pl.ds(tail, 8)], pe_t, spt)
            c2 = pltpu.async_copy(xp_hbm.at[idx_v.at[pl.ds(tail, 8)]], g_t,
                                  sgt)
            c1.wait()
            c2.wait()
            sum_rows(pe_t, g_t, r_t, 1)
            pltpu.sync_copy(r_t.at[pl.ds(0, C)],
                            out_hbm.at[b, pl.ds(tail * C, C)])

    return sc_kernel


def kernel(x, policy, mask_token, pos_embed):
    B, LV, C = x.shape
    L = policy.shape[1]
    Lp = ((L + _LANES - 1) // _LANES) * _LANES

    polp = jnp.pad(policy.astype(jnp.int32), ((0, 0), (0, Lp - L)))

    def _pack(a):
        r_, c_ = a.shape
        ab = a.astype(jnp.bfloat16).reshape(r_, c_ // 32, 2, _LANES)
        ab = ab.transpose(0, 1, 3, 2)
        return lax.bitcast_convert_type(ab, jnp.int32).reshape(r_, c_ // 2)

    pe = _pack(jnp.pad(pos_embed.reshape(L, C), ((0, 7), (0, 0))))
    xf = x.reshape(B * LV * C)
    mt = mask_token.reshape(C)

    sc = _build_sc_kernel(B, LV, C, L, Lp)
    out, _ = sc(xf, mt, polp, pe)
    return out.reshape(B, L, C)


# speedup vs baseline: 2.1538x; 1.3651x over previous
import functools

import jax
import jax.numpy as jnp
from jax import lax
from jax.experimental import pallas as pl
from jax.experimental.pallas import tpu as pltpu
from jax.experimental.pallas import tpu_sc as plsc

_NC, _NS = 2, 16
_CH = 32
_LANES = 16


def _build_sc_kernel(B, LV, C, L, Lp):
    TROWS = LV + 1
    n_chunks = L // _CH
    per_half = n_chunks // 2
    tail = n_chunks * _CH
    mesh = plsc.VectorSubcoreMesh(core_axis_name="c", subcore_axis_name="s")

    @functools.partial(
        pl.kernel,
        mesh=mesh,
        out_type=jax.ShapeDtypeStruct((B, L, C), jnp.float32),
        scratch_types=[
            pltpu.VMEM((Lp,), jnp.int32),
            pltpu.VMEM((Lp,), jnp.int32),
            pltpu.VMEM((_CH, C), jnp.float32),
            pltpu.VMEM((_CH, C), jnp.float32),
            pltpu.VMEM((_CH, C), jnp.float32),
            pltpu.VMEM((_CH, C), jnp.float32),
            pltpu.VMEM((8, C), jnp.float32),
            pltpu.VMEM((8, C), jnp.float32),
            pltpu.SemaphoreType.DMA,
            pltpu.SemaphoreType.DMA,
            pltpu.SemaphoreType.DMA,
            pltpu.SemaphoreType.DMA,
            pltpu.SemaphoreType.DMA,
            pltpu.SemaphoreType.DMA,
            pltpu.SemaphoreType.DMA,
            pltpu.SemaphoreType.DMA,
        ],
    )
    def sc_kernel(x_hbm, pol_hbm, pe_hbm, out_hbm, pol_v, idx_v,
                  pe0, g0, pe1, g1, pe_t, g_t,
                  spe0, sg0, so0, spe1, sg1, so1, spt, sgt):
        wid = lax.axis_index("s") * _NC + lax.axis_index("c")
        b = wid // 2
        h = wid % 2

        pltpu.sync_copy(pol_hbm.at[b], pol_v)

        sets = ((pe0, g0, spe0, sg0, so0), (pe1, g1, spe1, sg1, so1))
        starts = [(h * per_half + k) * _CH for k in range(per_half)]

        def issue_pe(k):
            pe_b, _, spe, _, _ = sets[k % 2]
            return pltpu.async_copy(pe_hbm.at[pl.ds(starts[k], _CH)], pe_b,
                                    spe)

        pe_pre = issue_pe(0)

        bbase = b * TROWS
        lanes = lax.iota(jnp.int32, _LANES)

        def _take(v, i):
            return v.at[i].get(mode="promise_in_bounds")

        def scan_body(i, carry):
            ch = pol_v[pl.ds(i * _LANES, _LANES)]
            cs = ch
            for d in (1, 2, 4, 8):
                shifted = _take(cs, jnp.maximum(lanes - d, 0))
                cs = cs + jnp.where(lanes >= d, shifted, 0)
            excl = cs - ch + carry
            idx_v[pl.ds(i * _LANES, _LANES)] = bbase + LV + ch * (excl - LV)
            return carry + _take(cs, jnp.full((_LANES,), _LANES - 1, jnp.int32))

        lax.fori_loop(0, Lp // _LANES, scan_body,
                      jnp.zeros((_LANES,), jnp.int32))

        def issue_g(k):
            _, g_b, _, sg, _ = sets[k % 2]
            return pltpu.async_copy(x_hbm.at[idx_v.at[pl.ds(starts[k], _CH)]],
                                    g_b, sg)

        def add_rows(pe_b, g_b, nrows):
            def row_body(r, _):
                for cc in range(C // _LANES):
                    sl = pl.ds(cc * _LANES, _LANES)
                    pe_b[r, sl] = pe_b[r, sl] + g_b[r, sl]
                return 0
            lax.fori_loop(0, nrows, row_body, 0)

        out_inflight = [None, None]
        in_h = (pe_pre, issue_g(0))
        for k in range(per_half):
            if k + 1 < per_half:
                nxt = (k + 1) % 2
                if out_inflight[nxt] is not None:
                    out_inflight[nxt].wait()
                    out_inflight[nxt] = None
                nxt_h = (issue_pe(k + 1), issue_g(k + 1))
            in_h[0].wait()
            in_h[1].wait()
            pe_b, g_b, _, _, so = sets[k % 2]
            add_rows(pe_b, g_b, _CH)
            out_inflight[k % 2] = pltpu.async_copy(
                pe_b, out_hbm.at[b, pl.ds(starts[k], _CH)], so)
            if k + 1 < per_half:
                in_h = nxt_h
        for o in out_inflight:
            if o is not None:
                o.wait()

        @pl.when(h == 1)
        def _():
            c1 = pltpu.async_copy(pe_hbm.at[pl.ds(tail, 1)],
                                  pe_t.at[pl.ds(0, 1)], spt)
            c2 = pltpu.async_copy(x_hbm.at[idx_v.at[pl.ds(tail, 8)]], g_t, sgt)
            c1.wait()
            c2.wait()
            add_rows(pe_t, g_t, 1)
            pltpu.sync_copy(pe_t.at[pl.ds(0, 1)],
                            out_hbm.at[b, pl.ds(tail, 1)])

    return sc_kernel


def kernel(x, policy, mask_token, pos_embed):
    B, LV, C = x.shape
    L = policy.shape[1]
    Lp = ((L + _LANES - 1) // _LANES) * _LANES

    polp = jnp.pad(policy.astype(jnp.int32), ((0, 0), (0, Lp - L)))
    x_ext = jnp.concatenate(
        [x, jnp.broadcast_to(mask_token, (B, 1, C))], axis=1
    ).reshape(B * (LV + 1), C)
    pe = pos_embed.reshape(L, C)

    sc = _build_sc_kernel(B, LV, C, L, Lp)
    return sc(x_ext, polp, pe)
